# Initial kernel scaffold; baseline (speedup 1.0000x reference)
import jax
import jax.numpy as jnp
from jax.experimental import pallas as pl


def kernel(x, proto, A, Bm, gains, top_k):
    raise NotImplementedError("write your pallas kernel here")


# speedup vs baseline: 87.0271x; 87.0271x over previous
import functools
import math

import numpy as np
import jax
import jax.numpy as jnp
from jax.experimental import pallas as pl
from jax.experimental.pallas import tpu as pltpu

_SIGN = np.array(0x80000000, dtype=np.uint32).view(np.int32).item()
_BITS = [np.array(1 << b, dtype=np.uint32).view(np.int32).item()
         for b in range(32)]


def _body(inv_scale, rr, kk_ref, x_ref, pT_ref, p_ref, af_ref, bfT_ref,
          gv_ref, o_ref):
    kk = kk_ref[0]
    x = x_ref[...]
    sim = jnp.dot(x, pT_ref[...],
                  preferred_element_type=jnp.float32) * inv_scale

    ki = jax.lax.bitcast_convert_type(sim, jnp.int32)
    key = jnp.where(ki >= 0, ki, jnp.int32(_SIGN) - ki)

    t = jnp.zeros((x.shape[0], 1), jnp.int32)
    for b in range(31, -1, -1):
        cand = jnp.bitwise_or(t, jnp.int32(_BITS[b]))
        scand = jnp.bitwise_xor(cand, jnp.int32(_SIGN))
        cnt = jnp.sum((key >= scand).astype(jnp.int32), axis=1, keepdims=True)
        t = jnp.where(cnt >= kk, cand, t)
    thr = jnp.bitwise_xor(t, jnp.int32(_SIGN))
    mask = key >= thr

    rowmax = jnp.max(sim, axis=1, keepdims=True)
    e = jnp.where(mask, jnp.exp(sim - rowmax), 0.0)
    s = jnp.sum(e, axis=1, keepdims=True)
    w = e / s

    gv = gv_ref[...]
    z = jnp.dot(w, af_ref[...], preferred_element_type=jnp.float32)
    c = jnp.dot(z * gv, bfT_ref[...], preferred_element_type=jnp.float32)
    sm = jnp.where(mask, c, 0.0)
    ctx = jnp.dot(sm, p_ref[...], preferred_element_type=jnp.float32)

    dsum = jnp.sum(gv) / jnp.float32(rr)
    dsum = jnp.where(dsum <= 0, jnp.float32(1.0), dsum)
    o_ref[...] = x + (jnp.float32(0.1) / dsum) * ctx


def kernel(x, proto, A, Bm, gains, top_k):
    B, T, D = x.shape
    N = proto.shape[0]
    NREL, _, R = A.shape
    kk = jnp.minimum(jnp.asarray(top_k, jnp.int32), min(128, N)).reshape(1)

    T2 = B * T
    TB = 128
    while T2 % TB:
        TB //= 2

    x2 = x.reshape(T2, D)
    protoT = proto.T
    af = jnp.transpose(A, (1, 0, 2)).reshape(N, NREL * R)
    bfT = jnp.transpose(Bm, (0, 2, 1)).reshape(NREL * R, N)
    gv = jnp.repeat(gains, R).reshape(1, NREL * R)

    out = pl.pallas_call(
        functools.partial(_body, 1.0 / math.sqrt(D), R),
        grid=(T2 // TB,),
        in_specs=[
            pl.BlockSpec(memory_space=pltpu.SMEM),
            pl.BlockSpec((TB, D), lambda i: (i, 0)),
            pl.BlockSpec((D, N), lambda i: (0, 0)),
            pl.BlockSpec((N, D), lambda i: (0, 0)),
            pl.BlockSpec((N, NREL * R), lambda i: (0, 0)),
            pl.BlockSpec((NREL * R, N), lambda i: (0, 0)),
            pl.BlockSpec((1, NREL * R), lambda i: (0, 0)),
        ],
        out_specs=pl.BlockSpec((TB, D), lambda i: (i, 0)),
        out_shape=jax.ShapeDtypeStruct((T2, D), jnp.float32),
        compiler_params=pltpu.CompilerParams(
            dimension_semantics=("arbitrary",)),
    )(kk, x2, protoT, proto, af, bfT, gv)
    return out.reshape(B, T, D)


# speedup vs baseline: 106.4920x; 1.2237x over previous
import functools
import math

import numpy as np
import jax
import jax.numpy as jnp
from jax.experimental import pallas as pl
from jax.experimental.pallas import tpu as pltpu

_SIGN = np.array(0x80000000, dtype=np.uint32).view(np.int32).item()
_BITS = [np.array(1 << b, dtype=np.uint32).view(np.int32).item()
         for b in range(32)]
_NSEL = 16


def _body(inv_scale, nrel, rank, kk_ref, g_ref, x_ref, p_ref, a_ref, b_ref,
          o_ref, pbf_ref, acat_ref, bcat_ref):
    @pl.when(pl.program_id(0) == 0)
    def _prep():
        pbf_ref[...] = p_ref[...].astype(jnp.bfloat16)
        for r in range(nrel):
            g = g_ref[r]
            acat_ref[:, r * rank:(r + 1) * rank] = (
                a_ref[r] * g).astype(jnp.bfloat16)
            bcat_ref[:, r * rank:(r + 1) * rank] = (
                b_ref[r]).astype(jnp.bfloat16)

    kk = kk_ref[0]
    x = x_ref[...]
    xb = x.astype(jnp.bfloat16)
    sim = jax.lax.dot_general(
        xb, pbf_ref[...], (((1,), (1,)), ((), ())),
        preferred_element_type=jnp.float32) * inv_scale

    ki = jax.lax.bitcast_convert_type(sim, jnp.int32)
    key = jnp.where(ki >= 0, ki, jnp.int32(_SIGN) - ki)

    t = jnp.zeros((x.shape[0], 1), jnp.int32)
    for b in range(31, 31 - _NSEL, -1):
        cand = jnp.bitwise_or(t, jnp.int32(_BITS[b]))
        scand = jnp.bitwise_xor(cand, jnp.int32(_SIGN))
        cnt = jnp.sum((key >= scand).astype(jnp.int32), axis=1, keepdims=True)
        t = jnp.where(cnt >= kk, cand, t)
    thr = jnp.bitwise_xor(t, jnp.int32(_SIGN))
    mask = key >= thr

    rowmax = jnp.max(sim, axis=1, keepdims=True)
    e = jnp.where(mask, jnp.exp(sim - rowmax), 0.0)
    s = jnp.sum(e, axis=1, keepdims=True)
    w = (e / s).astype(jnp.bfloat16)

    z = jnp.dot(w, acat_ref[...], preferred_element_type=jnp.float32)
    c = jax.lax.dot_general(
        z.astype(jnp.bfloat16), bcat_ref[...], (((1,), (1,)), ((), ())),
        preferred_element_type=jnp.float32)
    sm = jnp.where(mask, c, 0.0).astype(jnp.bfloat16)
    ctx = jnp.dot(sm, pbf_ref[...], preferred_element_type=jnp.float32)

    dsum = g_ref[0]
    for r in range(1, nrel):
        dsum = dsum + g_ref[r]
    dsum = jnp.where(dsum <= 0, jnp.float32(1.0), dsum)
    o_ref[...] = x + (jnp.float32(0.1) / dsum) * ctx


def kernel(x, proto, A, Bm, gains, top_k):
    B, T, D = x.shape
    N = proto.shape[0]
    NREL, _, R = A.shape
    kk = jnp.minimum(jnp.asarray(top_k, jnp.int32), min(128, N)).reshape(1)

    T2 = B * T
    TB = 128
    while T2 % TB:
        TB //= 2
    x2 = x.reshape(T2, D)

    out = pl.pallas_call(
        functools.partial(_body, 1.0 / math.sqrt(D), NREL, R),
        grid=(T2 // TB,),
        in_specs=[
            pl.BlockSpec(memory_space=pltpu.SMEM),
            pl.BlockSpec(memory_space=pltpu.SMEM),
            pl.BlockSpec((TB, D), lambda i: (i, 0)),
            pl.BlockSpec((N, D), lambda i: (0, 0)),
            pl.BlockSpec((NREL, N, R), lambda i: (0, 0, 0)),
            pl.BlockSpec((NREL, N, R), lambda i: (0, 0, 0)),
        ],
        out_specs=pl.BlockSpec((TB, D), lambda i: (i, 0)),
        out_shape=jax.ShapeDtypeStruct((T2, D), jnp.float32),
        scratch_shapes=[
            pltpu.VMEM((N, D), jnp.bfloat16),
            pltpu.VMEM((N, NREL * R), jnp.bfloat16),
            pltpu.VMEM((N, NREL * R), jnp.bfloat16),
        ],
        compiler_params=pltpu.CompilerParams(
            dimension_semantics=("arbitrary",)),
    )(kk, gains, x2, proto, A, Bm)
    return out.reshape(B, T, D)
